# Initial kernel scaffold; baseline (speedup 1.0000x reference)
import jax
import jax.numpy as jnp
from jax.experimental import pallas as pl


def kernel(x, edge_index, batch_index, W_rel0, b_rel0, W_root0, W_rel1, b_rel1, W_root1, W_rel2, b_rel2, W_root2, W_rel3, b_rel3, W_root3, W_out, b_out):
    raise NotImplementedError("write your pallas kernel here")


# speedup vs baseline: 2.8511x; 2.8511x over previous
import functools

import jax
import jax.numpy as jnp
from jax import lax
from jax.experimental import pallas as pl
from jax.experimental.pallas import tpu as pltpu, tpu_sc as plsc

N = 10000
E = 320000
F = 128
G = 64

NC = 2
NS = 16
NW = NC * NS

CH = 128
E_PAD = 327680
NCHUNK = E_PAD // (NW * CH)
NAGG = 10240
ZROWS = NAGG // NS

PCHUNKS = 96
PPT = PCHUNKS // NW
PROWS = 128


def _msgpass_body(h_hbm, srcw, dstw, zeros_hbm, out_hbm,
                  src_v, dst_v, rows_v, acc_sh, sem):
    c = lax.axis_index("c")
    s = lax.axis_index("s")
    w = c * NS + s
    pltpu.sync_copy(srcw.at[w], src_v)
    pltpu.sync_copy(dstw.at[w], dst_v)
    pltpu.sync_copy(zeros_hbm, acc_sh.at[pl.ds(s * ZROWS, ZROWS)])
    plsc.subcore_barrier()

    def body(j, carry):
        pltpu.async_copy(h_hbm.at[src_v.at[j]], rows_v, sem).wait()
        pltpu.sync_copy(rows_v, acc_sh.at[dst_v.at[j]], add=True)
        return carry

    lax.fori_loop(0, NCHUNK, body, 0)
    plsc.subcore_barrier()

    @pl.when(s < NS - 1)
    def _():
        pltpu.sync_copy(acc_sh.at[pl.ds(s * ZROWS, ZROWS)],
                        out_hbm.at[c, pl.ds(s * ZROWS, ZROWS)])

    @pl.when(s == NS - 1)
    def _():
        pltpu.sync_copy(acc_sh.at[pl.ds((NS - 1) * ZROWS, N - (NS - 1) * ZROWS)],
                        out_hbm.at[c, pl.ds((NS - 1) * ZROWS, N - (NS - 1) * ZROWS)])


@functools.cache
def _msgpass_kernel():
    return pl.kernel(
        _msgpass_body,
        out_type=jax.ShapeDtypeStruct((NC, N, F), jnp.float32),
        mesh=plsc.VectorSubcoreMesh(core_axis_name="c", subcore_axis_name="s",
                                    num_cores=NC, num_subcores=NS),
        scratch_types=[
            pltpu.VMEM((NCHUNK, CH), jnp.int32),
            pltpu.VMEM((NCHUNK, CH), jnp.int32),
            pltpu.VMEM((CH, F), jnp.float32),
            pltpu.VMEM_SHARED((NAGG, F), jnp.float32),
            pltpu.SemaphoreType.DMA,
        ],
    )


def _msgpass(*args):
    return _msgpass_kernel()(*args)


def _pool_body(h_hbm, nidw, bidw, zeros_hbm, ones_hbm, outp, outc,
               nid_v, bid_v, rows_v, ones_v, pool_sh, cnt_sh, sem):
    c = lax.axis_index("c")
    s = lax.axis_index("s")
    w = c * NS + s
    pltpu.sync_copy(nidw.at[w], nid_v)
    pltpu.sync_copy(bidw.at[w], bid_v)
    pltpu.sync_copy(ones_hbm, ones_v)

    @pl.when(s == 0)
    def _():
        pltpu.sync_copy(zeros_hbm.at[pl.ds(0, PROWS)], pool_sh)
        pltpu.sync_copy(zeros_hbm.at[pl.ds(PROWS, PROWS)], cnt_sh)

    plsc.subcore_barrier()

    def body(j, carry):
        pltpu.async_copy(h_hbm.at[nid_v.at[j]], rows_v, sem).wait()
        pltpu.sync_copy(rows_v, pool_sh.at[bid_v.at[j]], add=True)
        pltpu.sync_copy(ones_v, cnt_sh.at[bid_v.at[j]], add=True)
        return carry

    lax.fori_loop(0, PPT, body, 0)
    plsc.subcore_barrier()

    @pl.when(s == 0)
    def _():
        pltpu.sync_copy(pool_sh.at[pl.ds(0, G)], outp.at[c])
        pltpu.sync_copy(cnt_sh.at[pl.ds(0, G)], outc.at[c])


@functools.cache
def _pool_kernel():
    return pl.kernel(
        _pool_body,
        out_type=(jax.ShapeDtypeStruct((NC, G, F), jnp.float32),
                  jax.ShapeDtypeStruct((NC, G, F), jnp.float32)),
        mesh=plsc.VectorSubcoreMesh(core_axis_name="c", subcore_axis_name="s",
                                    num_cores=NC, num_subcores=NS),
        scratch_types=[
            pltpu.VMEM((PPT, CH), jnp.int32),
            pltpu.VMEM((PPT, CH), jnp.int32),
            pltpu.VMEM((CH, F), jnp.float32),
            pltpu.VMEM((CH, F), jnp.float32),
            pltpu.VMEM_SHARED((PROWS, F), jnp.float32),
            pltpu.VMEM_SHARED((PROWS, F), jnp.float32),
            pltpu.SemaphoreType.DMA,
        ],
    )


def _pool(*args):
    return _pool_kernel()(*args)


_RB = 1000


def _dense_body(a0, a1, h, wrel, brel, wroot, out):
    agg = a0[...] + a1[...]
    t = lax.dot_general(agg, wrel[...], (((1,), (1,)), ((), ())),
                        preferred_element_type=jnp.float32)
    t += lax.dot_general(h[...], wroot[...], (((1,), (1,)), ((), ())),
                         preferred_element_type=jnp.float32)
    t += brel[...]
    x = t
    out[...] = 0.5 * x * (1.0 + lax.erf(x * 0.7071067811865476))


def _dense_layer(a0, a1, h, wrel, brel, wroot):
    grid = (N // _RB,)
    row = pl.BlockSpec((_RB, F), lambda i: (i, 0))
    full = pl.BlockSpec((F, F), lambda i: (0, 0))
    return pl.pallas_call(
        _dense_body,
        grid=grid,
        in_specs=[row, row, row, full,
                  pl.BlockSpec((1, F), lambda i: (0, 0)), full],
        out_specs=row,
        out_shape=jax.ShapeDtypeStruct((N, F), jnp.float32),
    )(a0, a1, h, wrel, brel, wroot)


def _final_body(bout, p0, p1, c0, c1, wout, out):
    sums = p0[...] + p1[...]
    cnt = jnp.maximum(c0[...] + c1[...], 1.0)
    pooled = sums / cnt
    s = jnp.sum(pooled * wout[...], axis=1, keepdims=True) + bout[0]
    out[...] = jnp.broadcast_to(s, (G, F))


def _final(p0, p1, c0, c1, wout, bout):
    b = pl.BlockSpec((G, F), lambda: (0, 0))
    res = pl.pallas_call(
        _final_body,
        in_specs=[pl.BlockSpec(memory_space=pltpu.MemorySpace.SMEM),
                  b, b, b, b, pl.BlockSpec((1, F), lambda: (0, 0))],
        out_specs=b,
        out_shape=jax.ShapeDtypeStruct((G, F), jnp.float32),
    )(bout, p0, p1, c0, c1, wout)
    return res[:, :1]


def kernel(x, edge_index, batch_index,
           W_rel0, b_rel0, W_root0,
           W_rel1, b_rel1, W_root1,
           W_rel2, b_rel2, W_root2,
           W_rel3, b_rel3, W_root3,
           W_out, b_out):
    x = x.astype(jnp.float32)

    pad = E_PAD - E
    src = jnp.concatenate([edge_index[0], jnp.zeros((pad,), jnp.int32)])
    dst = jnp.concatenate([edge_index[1], jnp.full((pad,), N, jnp.int32)])
    srcw = src.reshape(NW, NCHUNK, CH)
    dstw = dst.reshape(NW, NCHUNK, CH)

    zeros_hbm = jnp.zeros((ZROWS, F), jnp.float32)
    ones_hbm = jnp.ones((CH, F), jnp.float32)

    ppad = PCHUNKS * CH - N
    nid = jnp.concatenate([jnp.arange(N, dtype=jnp.int32),
                           jnp.zeros((ppad,), jnp.int32)]).reshape(NW, PPT, CH)
    bid = jnp.concatenate([batch_index.astype(jnp.int32),
                           jnp.full((ppad,), PROWS - 1, jnp.int32)]
                          ).reshape(NW, PPT, CH)

    weights = [(W_rel0, b_rel0, W_root0), (W_rel1, b_rel1, W_root1),
               (W_rel2, b_rel2, W_root2), (W_rel3, b_rel3, W_root3)]

    h = x
    for (wrel, brel, wroot) in weights:
        parts = _msgpass(h, srcw, dstw, zeros_hbm)
        h = _dense_layer(parts[0], parts[1], h, wrel,
                         brel.reshape(1, F), wroot)

    psum, pcnt = _pool(h, nid, bid, jnp.zeros((2 * PROWS, F), jnp.float32),
                       ones_hbm)
    return _final(psum[0], psum[1], pcnt[0], pcnt[1], W_out, b_out)


# speedup vs baseline: 3.3691x; 1.1817x over previous
import functools

import jax
import jax.numpy as jnp
from jax import lax
from jax.experimental import pallas as pl
from jax.experimental.pallas import tpu as pltpu, tpu_sc as plsc

N = 10000
E = 320000
F = 128
G = 64

NC = 2
NS = 16
NW = NC * NS

CH = 128
E_PAD = 327680
TOTCH = E_PAD // CH
NCHUNK0 = 123
NCHUNK1 = TOTCH // NS - NCHUNK0
NAGG = 10240
DEPTH = 2
IDEPTH = 4
ZROWS = NAGG // NS

PCHUNKS = 96
PC0 = 5
PC1 = PCHUNKS // NS - PC0
PROWS = 128


def _msgpass_body(h_hbm, idxw, zeros_hbm, out_hbm,
                  iring, rows2, acc_sh, isem, rsem):
    c = lax.axis_index("c")
    s = lax.axis_index("s")
    base = jnp.where(c == 0, s * NCHUNK0, NS * NCHUNK0 + s * NCHUNK1)
    nch = jnp.where(c == 0, NCHUNK0, NCHUNK1)
    pltpu.sync_copy(zeros_hbm, acc_sh.at[pl.ds(s * ZROWS, ZROWS)])
    for k in range(IDEPTH):
        pltpu.async_copy(idxw.at[base + k], iring.at[k], isem)
    plsc.subcore_barrier()
    for k in range(DEPTH):
        pltpu.make_async_copy(idxw.at[base + k], iring.at[k], isem).wait()
        pltpu.async_copy(h_hbm.at[iring.at[k, 0]], rows2.at[k], rsem)

    def body(j, carry):
        rj = lax.rem(j, DEPTH)
        ij = lax.rem(j, IDEPTH)
        pltpu.make_async_copy(h_hbm.at[iring.at[ij, 0]],
                              rows2.at[rj], rsem).wait()
        pltpu.sync_copy(rows2.at[rj], acc_sh.at[iring.at[ij, 1]], add=True)

        @pl.when(j + IDEPTH < nch)
        def _():
            pltpu.async_copy(idxw.at[base + j + IDEPTH], iring.at[ij], isem)

        @pl.when(j + DEPTH < nch)
        def _():
            i2 = lax.rem(j + DEPTH, IDEPTH)
            pltpu.make_async_copy(idxw.at[base + j + DEPTH],
                                  iring.at[i2], isem).wait()
            pltpu.async_copy(h_hbm.at[iring.at[i2, 0]], rows2.at[rj], rsem)

        return carry

    lax.fori_loop(0, nch, body, 0)
    plsc.subcore_barrier()

    @pl.when(s < NS - 1)
    def _():
        pltpu.sync_copy(acc_sh.at[pl.ds(s * ZROWS, ZROWS)],
                        out_hbm.at[c, pl.ds(s * ZROWS, ZROWS)])

    @pl.when(s == NS - 1)
    def _():
        pltpu.sync_copy(
            acc_sh.at[pl.ds((NS - 1) * ZROWS, N - (NS - 1) * ZROWS)],
            out_hbm.at[c, pl.ds((NS - 1) * ZROWS, N - (NS - 1) * ZROWS)])


@functools.cache
def _msgpass_kernel():
    return pl.kernel(
        _msgpass_body,
        out_type=jax.ShapeDtypeStruct((NC, N, F), jnp.float32),
        mesh=plsc.VectorSubcoreMesh(core_axis_name="c", subcore_axis_name="s",
                                    num_cores=NC, num_subcores=NS),
        scratch_types=[
            pltpu.VMEM((IDEPTH, 2, CH), jnp.int32),
            pltpu.VMEM((DEPTH, CH, F), jnp.float32),
            pltpu.VMEM_SHARED((NAGG, F), jnp.float32),
            pltpu.SemaphoreType.DMA,
            pltpu.SemaphoreType.DMA,
        ],
    )


def _msgpass(*args):
    return _msgpass_kernel()(*args)


def _pool_body(h_hbm, idp, zeros_hbm, ones_hbm, outp, outc,
               iring, rows_v, ones_v, pool_sh, cnt_sh, sem):
    c = lax.axis_index("c")
    s = lax.axis_index("s")
    base = jnp.where(c == 0, s * PC0, NS * PC0 + s * PC1)
    nch = jnp.where(c == 0, PC0, PC1)
    pltpu.sync_copy(ones_hbm, ones_v)

    @pl.when(s == 0)
    def _():
        pltpu.sync_copy(zeros_hbm.at[pl.ds(0, PROWS)], pool_sh)
        pltpu.sync_copy(zeros_hbm.at[pl.ds(PROWS, PROWS)], cnt_sh)

    plsc.subcore_barrier()

    def body(j, carry):
        ij = lax.rem(j, 2)
        pltpu.sync_copy(idp.at[base + j], iring.at[ij])
        pltpu.async_copy(h_hbm.at[iring.at[ij, 0]], rows_v, sem).wait()
        pltpu.sync_copy(rows_v, pool_sh.at[iring.at[ij, 1]], add=True)
        pltpu.sync_copy(ones_v, cnt_sh.at[iring.at[ij, 1]], add=True)
        return carry

    lax.fori_loop(0, nch, body, 0)
    plsc.subcore_barrier()

    @pl.when(s == 0)
    def _():
        pltpu.sync_copy(pool_sh.at[pl.ds(0, G)], outp.at[c])
        pltpu.sync_copy(cnt_sh.at[pl.ds(0, G)], outc.at[c])


@functools.cache
def _pool_kernel():
    return pl.kernel(
        _pool_body,
        out_type=(jax.ShapeDtypeStruct((NC, G, F), jnp.float32),
                  jax.ShapeDtypeStruct((NC, G, F), jnp.float32)),
        mesh=plsc.VectorSubcoreMesh(core_axis_name="c", subcore_axis_name="s",
                                    num_cores=NC, num_subcores=NS),
        scratch_types=[
            pltpu.VMEM((2, 2, CH), jnp.int32),
            pltpu.VMEM((CH, F), jnp.float32),
            pltpu.VMEM((CH, F), jnp.float32),
            pltpu.VMEM_SHARED((PROWS, F), jnp.float32),
            pltpu.VMEM_SHARED((PROWS, F), jnp.float32),
            pltpu.SemaphoreType.DMA,
        ],
    )


def _pool(*args):
    return _pool_kernel()(*args)


_RB = 1000


def _dense_body(a0, a1, h, wrel, brel, wroot, out):
    agg = a0[...] + a1[...]
    t = lax.dot_general(agg, wrel[...], (((1,), (1,)), ((), ())),
                        preferred_element_type=jnp.float32)
    t += lax.dot_general(h[...], wroot[...], (((1,), (1,)), ((), ())),
                         preferred_element_type=jnp.float32)
    t += brel[...]
    x = t
    out[...] = 0.5 * x * (1.0 + lax.erf(x * 0.7071067811865476))


def _dense_layer(a0, a1, h, wrel, brel, wroot):
    grid = (N // _RB,)
    row = pl.BlockSpec((_RB, F), lambda i: (i, 0))
    full = pl.BlockSpec((F, F), lambda i: (0, 0))
    return pl.pallas_call(
        _dense_body,
        grid=grid,
        in_specs=[row, row, row, full,
                  pl.BlockSpec((1, F), lambda i: (0, 0)), full],
        out_specs=row,
        out_shape=jax.ShapeDtypeStruct((N, F), jnp.float32),
    )(a0, a1, h, wrel, brel, wroot)


def _final_body(bout, p0, p1, c0, c1, wout, out):
    sums = p0[...] + p1[...]
    cnt = jnp.maximum(c0[...] + c1[...], 1.0)
    pooled = sums / cnt
    sval = jnp.sum(pooled * wout[...], axis=1, keepdims=True) + bout[0]
    out[...] = jnp.broadcast_to(sval, (G, F))


def _final(p0, p1, c0, c1, wout, bout):
    b = pl.BlockSpec((G, F), lambda: (0, 0))
    res = pl.pallas_call(
        _final_body,
        in_specs=[pl.BlockSpec(memory_space=pltpu.MemorySpace.SMEM),
                  b, b, b, b, pl.BlockSpec((1, F), lambda: (0, 0))],
        out_specs=b,
        out_shape=jax.ShapeDtypeStruct((G, F), jnp.float32),
    )(bout, p0, p1, c0, c1, wout)
    return res[:, :1]


def kernel(x, edge_index, batch_index,
           W_rel0, b_rel0, W_root0,
           W_rel1, b_rel1, W_root1,
           W_rel2, b_rel2, W_root2,
           W_rel3, b_rel3, W_root3,
           W_out, b_out):
    x = x.astype(jnp.float32)

    pad = E_PAD - E
    src = jnp.concatenate([edge_index[0], jnp.zeros((pad,), jnp.int32)])
    dst = jnp.concatenate([edge_index[1],
                           N + (jnp.arange(pad, dtype=jnp.int32)
                                % (NAGG - N))])
    idxw = jnp.stack([src.reshape(TOTCH, CH),
                      dst.reshape(TOTCH, CH)], axis=1)

    zeros_hbm = jnp.zeros((ZROWS, F), jnp.float32)
    ones_hbm = jnp.ones((CH, F), jnp.float32)

    ppad = PCHUNKS * CH - N
    nid = jnp.concatenate([jnp.arange(N, dtype=jnp.int32),
                           jnp.zeros((ppad,), jnp.int32)])
    bid = jnp.concatenate([batch_index.astype(jnp.int32),
                           G + (jnp.arange(ppad, dtype=jnp.int32)
                                % (PROWS - G))])
    idp = jnp.stack([nid.reshape(PCHUNKS, CH),
                     bid.reshape(PCHUNKS, CH)], axis=1)

    weights = [(W_rel0, b_rel0, W_root0), (W_rel1, b_rel1, W_root1),
               (W_rel2, b_rel2, W_root2), (W_rel3, b_rel3, W_root3)]

    h = x
    for (wrel, brel, wroot) in weights:
        parts = _msgpass(h, idxw, zeros_hbm)
        h = _dense_layer(parts[0], parts[1], h, wrel,
                         brel.reshape(1, F), wroot)

    psum, pcnt = _pool(h, idp, jnp.zeros((2 * PROWS, F), jnp.float32),
                       ones_hbm)
    return _final(psum[0], psum[1], pcnt[0], pcnt[1], W_out, b_out)
